# Initial kernel scaffold; baseline (speedup 1.0000x reference)
import jax
import jax.numpy as jnp
from jax.experimental import pallas as pl


def kernel(x, edge_index, edge_weight, edge_attr, lin1_w, lin2_w, lin2_b, mlp_w1, mlp_b1, mlp_w2, mlp_b2, lin_w, lin_b):
    raise NotImplementedError("write your pallas kernel here")


# speedup vs baseline: 1.4233x; 1.4233x over previous
import functools
import math

import jax
import jax.numpy as jnp
from jax import lax
from jax.experimental import pallas as pl
from jax.experimental.pallas import tpu as pltpu
from jax.experimental.pallas import tpu_sc as plsc

HIDDEN = 128
NG = 50
CUTOFF = 10.0
N_NODES = 10000
N_EDGES = 320000

NC = 2
NS = 16
NW = NC * NS
EPT = N_EDGES // NW
K = 80
NCHUNK = EPT // K
PAD_NODES = 10240
ROWS_PER_TILE = PAD_NODES // NS
LOG2 = math.log(2.0)


def _ssp(v):
    return jnp.maximum(v, 0.0) + jnp.log1p(jnp.exp(-jnp.abs(v))) - LOG2


def _xh_body(x_ref, w_ref, o_ref):
    o_ref[...] = lax.dot_general(
        x_ref[...], w_ref[...], (((1,), (1,)), ((), ())),
        preferred_element_type=jnp.float32)


def _wf_body(ea_ref, ew_ref, w1_ref, b1_ref, w2_ref, b2_ref, o_ref):
    h = lax.dot_general(
        ea_ref[...], w1_ref[...], (((1,), (1,)), ((), ())),
        preferred_element_type=jnp.float32)
    h = _ssp(h + b1_ref[...])
    wf = lax.dot_general(
        h, w2_ref[...], (((1,), (1,)), ((), ())),
        preferred_element_type=jnp.float32) + b2_ref[...]
    c = 0.5 * (jnp.cos(ew_ref[...] * (math.pi / CUTOFF)) + 1.0)
    o_ref[...] = wf * c


def _gather_scatter(xh, src, dst, wf):
    mesh = plsc.VectorSubcoreMesh(
        core_axis_name="c", subcore_axis_name="s",
        num_cores=NC, num_subcores=NS)

    @functools.partial(
        pl.kernel,
        out_type=jax.ShapeDtypeStruct((NC, PAD_NODES, HIDDEN), jnp.float32),
        mesh=mesh,
        scratch_types=[
            pltpu.VMEM((K,), jnp.int32),
            pltpu.VMEM((K,), jnp.int32),
            pltpu.VMEM((K, HIDDEN), jnp.float32),
            pltpu.VMEM((K, HIDDEN), jnp.float32),
            pltpu.VMEM_SHARED((PAD_NODES, HIDDEN), jnp.float32),
            pltpu.SemaphoreType.DMA,
        ],
    )
    def k(xh_hbm, src_hbm, dst_hbm, wf_hbm, out_hbm,
          sidx, didx, rows, wfv, acc, sem):
        cid = lax.axis_index("c")
        sid = lax.axis_index("s")
        wid = sid * NC + cid

        def zrow(i, carry):
            for j in range(HIDDEN // 16):
                rows[i, pl.ds(j * 16, 16)] = jnp.zeros((16,), jnp.float32)
            return carry
        lax.fori_loop(0, K, zrow, 0)

        row0 = sid * ROWS_PER_TILE
        nfull = ROWS_PER_TILE // K

        def zcp(i, carry):
            pltpu.sync_copy(rows, acc.at[pl.ds(row0 + i * K, K)])
            return carry
        lax.fori_loop(0, nfull, zcp, 0)
        plsc.subcore_barrier()

        base0 = wid * EPT

        def chunk(cc, carry):
            base = base0 + cc * K
            pltpu.sync_copy(src_hbm.at[pl.ds(base, K)], sidx)
            pltpu.sync_copy(dst_hbm.at[pl.ds(base, K)], didx)
            pltpu.async_copy(xh_hbm.at[sidx], rows, sem).wait()
            pltpu.sync_copy(wf_hbm.at[pl.ds(base, K)], wfv)

            def mul(i, c2):
                for j in range(HIDDEN // 16):
                    sl = pl.ds(j * 16, 16)
                    rows[i, sl] = rows[i, sl] * wfv[i, sl]
                return c2
            lax.fori_loop(0, K, mul, 0)
            pltpu.sync_copy(rows, acc.at[didx], add=True)
            return carry
        lax.fori_loop(0, NCHUNK, chunk, 0)
        plsc.subcore_barrier()
        pltpu.sync_copy(acc.at[pl.ds(row0, ROWS_PER_TILE)],
                        out_hbm.at[cid, pl.ds(row0, ROWS_PER_TILE)])

    return k(xh, src, dst, wf)


def _final_body(p0_ref, p1_ref, w2_ref, b2_ref, lw_ref, lb_ref, o_ref):
    agg = p0_ref[...] + p1_ref[...]
    t = lax.dot_general(
        agg, w2_ref[...], (((1,), (1,)), ((), ())),
        preferred_element_type=jnp.float32) + b2_ref[...]
    t = _ssp(t)
    o_ref[...] = lax.dot_general(
        t, lw_ref[...], (((1,), (1,)), ((), ())),
        preferred_element_type=jnp.float32) + lb_ref[...]


def kernel(x, edge_index, edge_weight, edge_attr,
           lin1_w, lin2_w, lin2_b, mlp_w1, mlp_b1, mlp_w2, mlp_b2,
           lin_w, lin_b):
    src = edge_index[0]
    dst = edge_index[1]

    BN = 1000
    xh = pl.pallas_call(
        _xh_body,
        grid=(N_NODES // BN,),
        in_specs=[pl.BlockSpec((BN, HIDDEN), lambda i: (i, 0)),
                  pl.BlockSpec((HIDDEN, HIDDEN), lambda i: (0, 0))],
        out_specs=pl.BlockSpec((BN, HIDDEN), lambda i: (i, 0)),
        out_shape=jax.ShapeDtypeStruct((N_NODES, HIDDEN), jnp.float32),
    )(x, lin1_w)

    BE = 2000
    wf = pl.pallas_call(
        _wf_body,
        grid=(N_EDGES // BE,),
        in_specs=[pl.BlockSpec((BE, NG), lambda i: (i, 0)),
                  pl.BlockSpec((BE, 1), lambda i: (i, 0)),
                  pl.BlockSpec((HIDDEN, NG), lambda i: (0, 0)),
                  pl.BlockSpec((1, HIDDEN), lambda i: (0, 0)),
                  pl.BlockSpec((HIDDEN, HIDDEN), lambda i: (0, 0)),
                  pl.BlockSpec((1, HIDDEN), lambda i: (0, 0))],
        out_specs=pl.BlockSpec((BE, HIDDEN), lambda i: (i, 0)),
        out_shape=jax.ShapeDtypeStruct((N_EDGES, HIDDEN), jnp.float32),
    )(edge_attr, edge_weight.reshape(N_EDGES, 1), mlp_w1,
      mlp_b1.reshape(1, HIDDEN), mlp_w2, mlp_b2.reshape(1, HIDDEN))

    parts = _gather_scatter(xh, src, dst, wf)[:, :N_NODES]

    out = pl.pallas_call(
        _final_body,
        grid=(N_NODES // BN,),
        in_specs=[pl.BlockSpec((BN, HIDDEN), lambda i: (i, 0)),
                  pl.BlockSpec((BN, HIDDEN), lambda i: (i, 0)),
                  pl.BlockSpec((HIDDEN, HIDDEN), lambda i: (0, 0)),
                  pl.BlockSpec((1, HIDDEN), lambda i: (0, 0)),
                  pl.BlockSpec((HIDDEN, HIDDEN), lambda i: (0, 0)),
                  pl.BlockSpec((1, HIDDEN), lambda i: (0, 0))],
        out_specs=pl.BlockSpec((BN, HIDDEN), lambda i: (i, 0)),
        out_shape=jax.ShapeDtypeStruct((N_NODES, HIDDEN), jnp.float32),
    )(parts[0], parts[1], lin2_w, lin2_b.reshape(1, HIDDEN),
      lin_w, lin_b.reshape(1, HIDDEN))
    return out
